# Initial kernel scaffold; baseline (speedup 1.0000x reference)
import jax
import jax.numpy as jnp
from jax.experimental import pallas as pl


def kernel(box_weight):
    raise NotImplementedError("write your pallas kernel here")


# speedup vs baseline: 2.1684x; 2.1684x over previous
import jax
import jax.numpy as jnp
from jax.experimental import pallas as pl

TYPES_NUM = 100000
DIM = 128
ROWS = 1000


def _body(x_ref, c_ref, o_ref):
    x = x_ref[...]
    c_ref[...] = x[:, :DIM]
    o_ref[...] = jnp.maximum(x[:, DIM:], 0.0) + 1e-6


def kernel(box_weight):
    n = box_weight.shape[0]
    grid = (n // ROWS,)
    centers, offsets = pl.pallas_call(
        _body,
        grid=grid,
        in_specs=[pl.BlockSpec((ROWS, 2 * DIM), lambda i: (i, 0))],
        out_specs=[
            pl.BlockSpec((ROWS, DIM), lambda i: (i, 0)),
            pl.BlockSpec((ROWS, DIM), lambda i: (i, 0)),
        ],
        out_shape=[
            jax.ShapeDtypeStruct((n, DIM), jnp.float32),
            jax.ShapeDtypeStruct((n, DIM), jnp.float32),
        ],
    )(box_weight)
    return (centers, offsets)
